# baseline (speedup 1.0000x reference)
import functools

import jax
import jax.numpy as jnp
from jax import lax
from jax.experimental import pallas as pl
from jax.experimental.pallas import tpu as pltpu
from jax.experimental.pallas import tpu_sc as plsc

PAD_ID = 0

_NC = 2
_NS = 16
_NW = _NC * _NS
_L = 16
_KT = 800
_BB = 16
_SB = 40


@functools.cache
def _make_transpose(v, d):
  assert v % _KT == 0 and d % _L == 0 and _KT % 8 == 0
  nbl = v // _KT
  base_cnt, extra = divmod(nbl, _NW)
  mesh = plsc.VectorSubcoreMesh(core_axis_name="c", subcore_axis_name="s")

  @functools.partial(
      pl.kernel,
      out_type=jax.ShapeDtypeStruct((v, d), jnp.float32),
      mesh=mesh,
      scratch_types=[
          pltpu.VMEM((d, _KT), jnp.float32),
          pltpu.VMEM((_KT, d), jnp.float32),
          pltpu.SemaphoreType.DMA,
      ],
      compiler_params=pltpu.CompilerParams(
          use_tc_tiling_on_sc=False, needs_layout_passes=False
      ),
  )
  def transpose_k(tab_hbm, out_hbm, inb, outb, ws):
    wid = lax.axis_index("s") * _NC + lax.axis_index("c")
    my_n = base_cnt + jnp.where(wid < extra, 1, 0)
    iota = lax.iota(jnp.int32, _L)
    row_vecs = [iota + jg * _L for jg in range(d // _L)]

    def block(k, carry):
      c0 = (wid + k * _NW) * _KT
      pltpu.sync_copy(tab_hbm.at[:, pl.ds(c0, _KT)], inb)

      def row(t, cin):
        tcol = jnp.full((_L,), t, jnp.int32)
        for jg in range(d // _L):
          src = plsc.load_gather(inb, [row_vecs[jg], tcol])
          outb[t, pl.ds(jg * _L, _L)] = src
        return cin

      lax.fori_loop(0, _KT, row, 0)
      pltpu.sync_copy(outb, out_hbm.at[pl.ds(c0, _KT)])
      return carry

    lax.fori_loop(0, my_n, block, 0)

  return transpose_k


@functools.cache
def _make_gather(n_rows, v, d, max_pos, batch, seq):
  assert batch % (_NW * _BB) == 0 and seq % _SB == 0 and _SB % 8 == 0
  b_per_w = batch // _NW
  nb = b_per_w // _BB
  ns = seq // _SB
  ntok = _BB * _SB
  ng = ntok // 128
  assert ntok % 128 == 0
  mesh = plsc.VectorSubcoreMesh(core_axis_name="c", subcore_axis_name="s")

  @functools.partial(
      pl.kernel,
      out_type=jax.ShapeDtypeStruct((seq * d, batch), jnp.float32),
      mesh=mesh,
      scratch_types=[
          pltpu.VMEM((ntok,), jnp.int32),
          pltpu.VMEM((_SB, _BB), jnp.float32),
          pltpu.VMEM((ntok, d), jnp.float32),
          pltpu.VMEM((_SB * d, _BB), jnp.float32),
          pltpu.VMEM((max_pos, d), jnp.float32),
          pltpu.SemaphoreType.DMA,
          pltpu.SemaphoreType.DMA,
      ],
      compiler_params=pltpu.CompilerParams(
          use_tc_tiling_on_sc=False, needs_layout_passes=False
      ),
  )
  def gather_k(tok_hbm, table_hbm, pos_hbm, out_hbm,
               idxb, maskt, rows, outs, posbuf, gs, ws):
    wid = lax.axis_index("s") * _NC + lax.axis_index("c")
    b_base = wid * b_per_w
    iota16 = lax.iota(jnp.int32, _L)
    iota_rows = iota16 * _SB

    pltpu.sync_copy(pos_hbm, posbuf)

    def block(bi, si):
      b0 = b_base + bi * _BB
      s0 = si * _SB
      for i in range(_BB):
        pltpu.sync_copy(
            tok_hbm.at[pl.ds((b0 + i) * seq + s0, _SB)],
            idxb.at[pl.ds(i * _SB, _SB)],
        )
      cps = [
          pltpu.async_copy(
              table_hbm.at[idxb.at[pl.ds(g * 128, 128)]],
              rows.at[pl.ds(g * 128, 128)],
              gs,
          )
          for g in range(ng)
      ]
      for g in range(ntok // _L):
        tv = idxb[pl.ds(g * _L, _L)]
        fl = jnp.full((_L,), g * _L) + iota16
        st = lax.rem(fl, _SB)
        bt = lax.div(fl, _SB)
        mv = jnp.where(tv != PAD_ID, 1.0, 0.0).astype(jnp.float32)
        plsc.store_scatter(maskt, [st, bt], mv)
      for cp in cps:
        cp.wait()

      def srow(sl, carry):
        mvec = maskt[sl, :]
        base = sl * d
        rvec = iota_rows + jnp.full((_L,), sl, jnp.int32)
        for jg in range(d // _L):
          pv = posbuf[s0 + sl, pl.ds(jg * _L, _L)]
          for jj in range(_L):
            j = jg * _L + jj
            src = plsc.load_gather(rows, [rvec, jnp.full((_L,), j, jnp.int32)])
            outs[base + j, :] = src * mvec + jnp.full((_L,), pv[jj])
        return carry

      lax.fori_loop(0, _SB, srow, 0)
      pltpu.sync_copy(outs, out_hbm.at[pl.ds(s0 * d, _SB * d), pl.ds(b0, _BB)])

    def outer(i):
      block(lax.div(i, ns), lax.rem(i, ns))

    pl.loop(0, nb * ns)(outer)

  return gather_k


def kernel(tokens, embed_weight, pos_weight):
  batch, seq = tokens.shape
  v, d = embed_weight.shape
  max_pos = pos_weight.shape[0]
  tok_flat = tokens.astype(jnp.int32).reshape(-1)
  packed = _make_transpose(v, d)(embed_weight.T)
  out_t = _make_gather(tok_flat.shape[0], v, d, max_pos, batch, seq)(
      tok_flat, packed, pos_weight
  )
  return out_t.reshape(seq, d, batch).transpose(2, 0, 1)


# speedup vs baseline: 1.2400x; 1.2400x over previous
import functools

import jax
import jax.numpy as jnp
from jax import lax
from jax.experimental import pallas as pl
from jax.experimental.pallas import tpu as pltpu
from jax.experimental.pallas import tpu_sc as plsc

PAD_ID = 0

_NC = 2
_NS = 16
_NW = _NC * _NS
_L = 16
_KT = 800
_KTP = 801
_BB = 16
_SB = 40


@functools.cache
def _make_transpose(v, d):
  assert v % _KT == 0 and d % _L == 0 and _KT % 8 == 0
  nbl = v // _KT
  base_cnt, extra = divmod(nbl, _NW)
  mesh = plsc.VectorSubcoreMesh(core_axis_name="c", subcore_axis_name="s")

  @functools.partial(
      pl.kernel,
      out_type=jax.ShapeDtypeStruct((v, d), jnp.float32),
      mesh=mesh,
      scratch_types=[
          pltpu.VMEM((d, _KTP), jnp.float32),
          pltpu.VMEM((_KT, d), jnp.float32),
          pltpu.SemaphoreType.DMA,
      ],
      compiler_params=pltpu.CompilerParams(
          use_tc_tiling_on_sc=False, needs_layout_passes=False
      ),
  )
  def transpose_k(tab_hbm, out_hbm, inb, outb, ws):
    wid = lax.axis_index("s") * _NC + lax.axis_index("c")
    my_n = base_cnt + jnp.where(wid < extra, 1, 0)
    iota = lax.iota(jnp.int32, _L)
    row_vecs = [iota + jg * _L for jg in range(d // _L)]

    def block(k, carry):
      c0 = (wid + k * _NW) * _KT
      pltpu.sync_copy(tab_hbm.at[:, pl.ds(c0, _KT)], inb.at[:, pl.ds(0, _KT)])

      def row(t, cin):
        for u in range(2):
          tcol = jnp.full((_L,), 2 * t + u, jnp.int32)
          for jg in range(d // _L):
            src = plsc.load_gather(inb, [row_vecs[jg], tcol])
            outb[2 * t + u, pl.ds(jg * _L, _L)] = src
        return cin

      lax.fori_loop(0, _KT // 2, row, 0)
      pltpu.sync_copy(outb, out_hbm.at[pl.ds(c0, _KT)])
      return carry

    lax.fori_loop(0, my_n, block, 0)

  return transpose_k


@functools.cache
def _make_gather(n_rows, v, d, max_pos, batch, seq):
  assert batch % (_NW * _BB) == 0 and seq % _SB == 0 and _SB % 8 == 0
  bbp = _BB + 1
  b_per_w = batch // _NW
  nb = b_per_w // _BB
  ns = seq // _SB
  ntok = _BB * _SB
  mesh = plsc.VectorSubcoreMesh(core_axis_name="c", subcore_axis_name="s")

  @functools.partial(
      pl.kernel,
      out_type=jax.ShapeDtypeStruct((seq * d, batch), jnp.float32),
      mesh=mesh,
      scratch_types=[
          pltpu.VMEM((_BB, _SB), jnp.int32),
          pltpu.VMEM((ntok, d), jnp.float32),
          pltpu.VMEM((_SB * d, bbp), jnp.float32),
          pltpu.VMEM((max_pos, d), jnp.float32),
          pltpu.SemaphoreType.DMA,
          pltpu.SemaphoreType.DMA,
      ],
      compiler_params=pltpu.CompilerParams(
          use_tc_tiling_on_sc=False, needs_layout_passes=False
      ),
  )
  def gather_k(tok_hbm, table_hbm, pos_hbm, out_hbm,
               idxb, rows, outs, posbuf, gs, ws):
    wid = lax.axis_index("s") * _NC + lax.axis_index("c")
    b_base = wid * b_per_w
    iota16 = lax.iota(jnp.int32, _L)

    pltpu.sync_copy(pos_hbm, posbuf)

    def block(bi, si):
      b0 = b_base + bi * _BB
      s0 = si * _SB
      pltpu.sync_copy(
          tok_hbm.at[pl.ds(b0, _BB), pl.ds(s0, _SB)], idxb
      )
      cps = [
          pltpu.async_copy(
              table_hbm.at[idxb.at[i]], rows.at[pl.ds(i * _SB, _SB)], gs
          )
          for i in range(_BB)
      ]
      for cp in cps:
        cp.wait()

      def srow(sl, carry):
        tvec = plsc.load_gather(idxb, [iota16, jnp.full((_L,), sl, jnp.int32)])
        mvec = jnp.where(tvec != PAD_ID, 1.0, 0.0).astype(jnp.float32)
        pos4 = [posbuf[s0 + sl, pl.ds(jg * _L, _L)] for jg in range(d // _L)]
        rbase = sl * d
        for b in range(_BB):
          m = jnp.full((_L,), mvec[b])
          trow = b * _SB + sl
          for jg in range(d // _L):
            val = rows[trow, pl.ds(jg * _L, _L)] * m + pos4[jg]
            plsc.store_scatter(
                outs,
                [jnp.full((_L,), rbase + jg * _L, jnp.int32) + iota16,
                 jnp.full((_L,), b, jnp.int32)],
                val,
            )
        return carry

      lax.fori_loop(0, _SB, srow, 0)
      pltpu.sync_copy(
          outs.at[:, pl.ds(0, _BB)],
          out_hbm.at[pl.ds(s0 * d, _SB * d), pl.ds(b0, _BB)],
      )

    def outer(i):
      block(lax.div(i, ns), lax.rem(i, ns))

    pl.loop(0, nb * ns)(outer)

  return gather_k


def kernel(tokens, embed_weight, pos_weight):
  batch, seq = tokens.shape
  v, d = embed_weight.shape
  max_pos = pos_weight.shape[0]
  tok32 = tokens.astype(jnp.int32)
  packed = _make_transpose(v, d)(embed_weight.T)
  out_t = _make_gather(batch * seq, v, d, max_pos, batch, seq)(
      tok32, packed, pos_weight
  )
  return out_t.reshape(seq, d, batch).transpose(2, 0, 1)


# speedup vs baseline: 6.8798x; 5.5482x over previous
import functools

import jax
import jax.numpy as jnp
from jax import lax
from jax.experimental import pallas as pl
from jax.experimental.pallas import tpu as pltpu
from jax.experimental.pallas import tpu_sc as plsc

PAD_ID = 0

_NC = 2
_NS = 16
_NW = _NC * _NS
_L = 16
_BB = 16
_SB = 40


@functools.cache
def _make_gather(v, d, max_pos, batch, seq):
  assert batch % (_NW * _BB) == 0 and seq % _SB == 0 and _SB % 8 == 0
  bbp = _BB + 1
  b_per_w = batch // _NW
  nb = b_per_w // _BB
  ns = seq // _SB
  ntok = _BB * _SB
  mesh = plsc.VectorSubcoreMesh(core_axis_name="c", subcore_axis_name="s")

  @functools.partial(
      pl.kernel,
      out_type=jax.ShapeDtypeStruct((seq * d, batch), jnp.float32),
      mesh=mesh,
      scratch_types=[
          pltpu.VMEM((_BB, _SB), jnp.int32),
          pltpu.VMEM((ntok, d), jnp.float32),
          pltpu.VMEM((_SB * d, bbp), jnp.float32),
          pltpu.VMEM((max_pos, d), jnp.float32),
          pltpu.SemaphoreType.DMA,
          pltpu.SemaphoreType.DMA,
      ],
      compiler_params=pltpu.CompilerParams(
          use_tc_tiling_on_sc=False, needs_layout_passes=False
      ),
  )
  def gather_k(tok_hbm, table_hbm, pos_hbm, out_hbm,
               idxb, rows, outs, posbuf, gs, ws):
    wid = lax.axis_index("s") * _NC + lax.axis_index("c")
    b_base = wid * b_per_w
    iota16 = lax.iota(jnp.int32, _L)

    pltpu.sync_copy(pos_hbm, posbuf)

    def block(bi, si):
      b0 = b_base + bi * _BB
      s0 = si * _SB
      pltpu.sync_copy(
          tok_hbm.at[pl.ds(b0, _BB), pl.ds(s0, _SB)], idxb
      )
      cps = [
          pltpu.async_copy(
              table_hbm.at[idxb.at[i]], rows.at[pl.ds(i * _SB, _SB)], gs
          )
          for i in range(_BB)
      ]
      for cp in cps:
        cp.wait()

      @functools.partial(plsc.parallel_loop, 0, _SB)
      def srow(sl):
        tvec = plsc.load_gather(idxb, [iota16, jnp.full((_L,), sl, jnp.int32)])
        mvec = jnp.where(tvec != PAD_ID, 1.0, 0.0).astype(jnp.float32)
        pos4 = [posbuf[s0 + sl, pl.ds(jg * _L, _L)] for jg in range(d // _L)]
        rbase = sl * d
        for b in range(_BB):
          m = jnp.full((_L,), mvec[b])
          trow = b * _SB + sl
          for jg in range(d // _L):
            val = rows[trow, pl.ds(jg * _L, _L)] * m + pos4[jg]
            plsc.store_scatter(
                outs,
                [jnp.full((_L,), rbase + jg * _L, jnp.int32) + iota16,
                 jnp.full((_L,), b, jnp.int32)],
                val,
            )

      pltpu.sync_copy(
          outs.at[:, pl.ds(0, _BB)],
          out_hbm.at[pl.ds(s0 * d, _SB * d), pl.ds(b0, _BB)],
      )

    def outer(i):
      block(lax.div(i, ns), lax.rem(i, ns))

    pl.loop(0, nb * ns)(outer)

  return gather_k


def kernel(tokens, embed_weight, pos_weight):
  batch, seq = tokens.shape
  v, d = embed_weight.shape
  max_pos = pos_weight.shape[0]
  tok32 = tokens.astype(jnp.int32)
  out_t = _make_gather(v, d, max_pos, batch, seq)(
      tok32, embed_weight, pos_weight
  )
  return out_t.reshape(seq, d, batch).transpose(2, 0, 1)
